# baseline (speedup 1.0000x reference)
import jax
import jax.numpy as jnp
from jax import lax
from jax.experimental import pallas as pl
from jax.experimental.pallas import tpu as pltpu
from jax.experimental.pallas import tpu_sc as plsc

_NEG_INF = float("-inf")
_POS_INF = float("inf")
_N = 1024
_CHUNKS = _N // 16
_UNROLL = 8
_NBUF = 4


def _sc_body(d2_hbm, eps_hbm, it_hbm, dist_hbm, asg_hbm,
             price_v, asg_v, inv_v, maxi_v, win_v, pend_v, idl_v,
             row_0, row_1, row_2, row_3, scal_f, scal_i,
             sem_0, sem_1, sem_2, sem_3):
    c = lax.axis_index("c")
    s = lax.axis_index("s")
    b = s * 2 + c
    bufs = (row_0, row_1, row_2, row_3)
    sems = (sem_0, sem_1, sem_2, sem_3)

    @pl.when(s < 8)
    def _run():
        pltpu.sync_copy(eps_hbm, scal_f)
        pltpu.sync_copy(it_hbm, scal_i)
        eps = scal_f[...][0]
        niter = scal_i[...][0]

        lanes = lax.iota(jnp.int32, 16)
        zf16 = jnp.zeros((16,), jnp.float32)
        neg1 = jnp.full((16,), -1, jnp.int32)
        ninf16 = jnp.full((16,), _NEG_INF, jnp.float32)
        pinf16 = jnp.full((16,), _POS_INF, jnp.float32)
        big16 = jnp.full((16,), 1 << 30, jnp.int32)
        lane0 = lanes == 0
        rowbase = b * _N

        def _init(g, carry):
            for u in range(_UNROLL):
                ds = pl.ds(g * (16 * _UNROLL) + u * 16, 16)
                price_v[ds] = zf16
                asg_v[ds] = neg1
                inv_v[ds] = neg1
                pend_v[ds] = zf16
            return carry

        lax.fori_loop(0, _CHUNKS // _UNROLL, _init, 0)

        def _get_rid(q):
            qv = jnp.broadcast_to(q, (16,))
            return plsc.load_gather(idl_v, [qv])[0]

        def _start(q, buf, sem):
            src = d2_hbm.at[rowbase + _get_rid(q)]
            pltpu.make_async_copy(src, buf, sem).start()

        def _wait(buf, sem):
            pltpu.make_async_copy(d2_hbm.at[0], buf, sem).wait()

        def _process(buf, rid):
            def scan_c(g, car):
                out = []
                base = g * (16 * _UNROLL)
                for u in range(_UNROLL):
                    v1l, v2l, j1l = car[3 * u:3 * u + 3]
                    ds = pl.ds(base + u * 16, 16)
                    v = buf[ds] + price_v[ds]
                    cols = lanes + (base + u * 16)
                    lt1 = v < v1l
                    v2l = jnp.minimum(v2l, jnp.maximum(v1l, v))
                    v1l = jnp.minimum(v1l, v)
                    j1l = jnp.where(lt1, cols, j1l)
                    out += [v1l, v2l, j1l]
                return tuple(out)

            acc = lax.fori_loop(0, _CHUNKS // _UNROLL, scan_c,
                                (pinf16, pinf16, big16) * _UNROLL)

            def merge(a, bb):
                v1 = jnp.minimum(a[0], bb[0])
                j1 = jnp.minimum(jnp.where(a[0] == v1, a[2], big16),
                                 jnp.where(bb[0] == v1, bb[2], big16))
                ea = (a[0] == v1) & (a[2] == j1)
                eb = (bb[0] == v1) & (bb[2] == j1)
                v2 = jnp.minimum(jnp.where(ea, a[1], a[0]),
                                 jnp.where(eb, bb[1], bb[0]))
                return v1, v2, j1

            m0 = merge(acc[0:3], acc[3:6])
            m1 = merge(acc[6:9], acc[9:12])
            v1l, v2l, j1l = merge(m0, m1)

            gv1 = jnp.min(v1l)
            gbj = jnp.min(jnp.where(v1l == gv1, j1l, big16))
            gv2 = jnp.min(jnp.where(j1l == gbj, v2l, v1l))
            incr = (gv2 - gv1) + eps

            gbj_v = jnp.broadcast_to(gbj, (16,))
            rid_v = jnp.broadcast_to(rid, (16,))
            dval = plsc.load_gather(buf, [gbj_v])
            plsc.store_scatter(pend_v, [rid_v], dval, mask=lane0)
            cur = plsc.load_gather(maxi_v, [gbj_v])
            incr_v = jnp.broadcast_to(incr, (16,))
            upd = (incr_v > cur) & lane0
            plsc.store_scatter(maxi_v, [gbj_v], incr_v, mask=upd)
            plsc.store_scatter(win_v, [gbj_v], rid_v, mask=upd)

        def _iter(t, carry):
            def ph_a(g, cnt):
                for u in range(_UNROLL):
                    base = g * (16 * _UNROLL) + u * 16
                    ds = pl.ds(base, 16)
                    maxi_v[ds] = ninf16
                    win_v[ds] = neg1
                    mask = asg_v[ds] < 0
                    pos = plsc.cumsum(mask.astype(jnp.int32))
                    plsc.store_scatter(idl_v, [cnt + pos - 1], lanes + base,
                                       mask=mask)
                    cnt = cnt + pos[15]
                return cnt

            nb = lax.fori_loop(0, _CHUNKS // _UNROLL, ph_a, jnp.int32(0))

            for u in range(_NBUF):
                @pl.when(u < nb)
                def _(u=u):
                    _start(jnp.int32(u), bufs[u], sems[u])

            def ph_b(k, carry2):
                for u in range(_NBUF):
                    q = k * _NBUF + u

                    @pl.when(q < nb)
                    def _(q=q, u=u):
                        _wait(bufs[u], sems[u])
                        _process(bufs[u], _get_rid(q))

                        @pl.when(q + _NBUF < nb)
                        def _(q=q, u=u):
                            _start(q + _NBUF, bufs[u], sems[u])

                return carry2

            lax.fori_loop(0, (nb + _NBUF - 1) // _NBUF, ph_b, 0)

            def ph_c(g, carry3):
                for u in range(_UNROLL):
                    base = g * (16 * _UNROLL) + u * 16
                    ds = pl.ds(base, 16)
                    mi = maxi_v[ds]
                    wn = win_v[ds]
                    hb = mi > _NEG_INF
                    pr = price_v[ds]
                    price_v[ds] = jnp.where(hb, pr + mi, pr)
                    prev = inv_v[ds]
                    mask_e = hb & (prev >= 0)
                    plsc.store_scatter(asg_v, [jnp.where(mask_e, prev, 0)],
                                       neg1, mask=mask_e)
                    plsc.store_scatter(asg_v, [jnp.where(hb, wn, 0)],
                                       lanes + base, mask=hb)
                    inv_v[ds] = jnp.where(hb, wn, prev)
                return carry3

            lax.fori_loop(0, _CHUNKS // _UNROLL, ph_c, 0)
            return carry

        lax.fori_loop(0, niter, _iter, 0)

        def _fin(g, carry):
            for u in range(_UNROLL):
                ds = pl.ds(g * (16 * _UNROLL) + u * 16, 16)
                pend_v[ds] = jnp.where(asg_v[ds] >= 0, pend_v[ds], 0.0)
            return carry

        lax.fori_loop(0, _CHUNKS // _UNROLL, _fin, 0)
        pltpu.sync_copy(pend_v, dist_hbm.at[b])
        pltpu.sync_copy(asg_v, asg_hbm.at[b])


def kernel(input1, input2, eps, iters):
    B, n, _ = input1.shape
    m = input2.shape[1]
    s1 = jnp.sum(input1 * input1, axis=-1)[:, :, None]
    s2 = jnp.sum(input2 * input2, axis=-1)[:, None, :]
    cross = jnp.einsum('bnd,bmd->bnm', input1, input2)
    d2 = jnp.maximum(s1 + s2 - 2.0 * cross, 0.0)

    eps_arr = jnp.full((16,), eps, jnp.float32)
    it_arr = jnp.full((16,), iters, jnp.int32)
    mesh = plsc.VectorSubcoreMesh(core_axis_name="c", subcore_axis_name="s",
                                  num_cores=2, num_subcores=16)
    run = pl.kernel(
        _sc_body,
        out_type=[
            jax.ShapeDtypeStruct((B, n), jnp.float32),
            jax.ShapeDtypeStruct((B, n), jnp.int32),
        ],
        mesh=mesh,
        compiler_params=pltpu.CompilerParams(needs_layout_passes=False),
        scratch_types=[
            pltpu.VMEM((n,), jnp.float32),
            pltpu.VMEM((n,), jnp.int32),
            pltpu.VMEM((n,), jnp.int32),
            pltpu.VMEM((n,), jnp.float32),
            pltpu.VMEM((n,), jnp.int32),
            pltpu.VMEM((n,), jnp.float32),
            pltpu.VMEM((n,), jnp.int32),
            pltpu.VMEM((m,), jnp.float32),
            pltpu.VMEM((m,), jnp.float32),
            pltpu.VMEM((m,), jnp.float32),
            pltpu.VMEM((m,), jnp.float32),
            pltpu.VMEM((16,), jnp.float32),
            pltpu.VMEM((16,), jnp.int32),
            pltpu.SemaphoreType.DMA,
            pltpu.SemaphoreType.DMA,
            pltpu.SemaphoreType.DMA,
            pltpu.SemaphoreType.DMA,
        ],
    )
    dist, asg = run(d2.reshape(B * n, m), eps_arr, it_arr)
    return dist, asg


# speedup vs baseline: 3.7939x; 3.7939x over previous
import jax
import jax.numpy as jnp
from jax import lax
from jax.experimental import pallas as pl
from jax.experimental.pallas import tpu as pltpu
from jax.experimental.pallas import tpu_sc as plsc

_NEG_INF = float("-inf")
_POS_INF = float("inf")
_N = 1024
_HN = _N // 2
_CHUNKS = _N // 16
_HCHUNKS = _HN // 16
_UNROLL = 4
_NBUF = 4


def _sc_body(d2_hbm, eps_hbm, it_hbm, dist_hbm, asg_hbm,
             price_v, asg_v, inv_v, maxi_v, win_v, pend_v, idl_v,
             mx1_v, wn1_v,
             row_0, row_1, row_2, row_3, scal_f, scal_i,
             shf, shi,
             sem_0, sem_1, sem_2, sem_3):
    c = lax.axis_index("c")
    s = lax.axis_index("s")
    sl = s // 2
    h = s % 2
    b = c * 8 + sl
    bufs = (row_0, row_1, row_2, row_3)
    sems = (sem_0, sem_1, sem_2, sem_3)

    pltpu.sync_copy(eps_hbm, scal_f)
    pltpu.sync_copy(it_hbm, scal_i)
    eps = scal_f[...][0]
    niter = scal_i[...][0]

    lanes = lax.iota(jnp.int32, 16)
    zf16 = jnp.zeros((16,), jnp.float32)
    neg1 = jnp.full((16,), -1, jnp.int32)
    ninf16 = jnp.full((16,), _NEG_INF, jnp.float32)
    pinf16 = jnp.full((16,), _POS_INF, jnp.float32)
    big16 = jnp.full((16,), 1 << 30, jnp.int32)
    lane0 = lanes == 0
    rowbase = b * _N
    hbase = h * _HN

    def _init(g, carry):
        for u in range(_UNROLL):
            ds = pl.ds(g * (16 * _UNROLL) + u * 16, 16)
            price_v[ds] = zf16
            asg_v[ds] = neg1
            inv_v[ds] = neg1
            pend_v[ds] = zf16
        return carry

    lax.fori_loop(0, _CHUNKS // _UNROLL, _init, 0)

    def _get_rid(q):
        qv = jnp.broadcast_to(q, (16,))
        return plsc.load_gather(idl_v, [qv])[0]

    def _start(q, buf, sem):
        src = d2_hbm.at[rowbase + _get_rid(q)]
        pltpu.make_async_copy(src, buf, sem).start()

    def _wait(buf, sem):
        pltpu.make_async_copy(d2_hbm.at[0], buf, sem).wait()

    def _process(buf, rid):
        def scan_c(g, car):
            out = []
            base = g * (16 * _UNROLL)
            for u in range(_UNROLL):
                v1l, v2l, j1l = car[3 * u:3 * u + 3]
                ds = pl.ds(base + u * 16, 16)
                v = buf[ds] + price_v[ds]
                cols = lanes + (base + u * 16)
                lt1 = v < v1l
                v2l = jnp.minimum(v2l, jnp.maximum(v1l, v))
                v1l = jnp.minimum(v1l, v)
                j1l = jnp.where(lt1, cols, j1l)
                out += [v1l, v2l, j1l]
            return tuple(out)

        acc = lax.fori_loop(0, _CHUNKS // _UNROLL, scan_c,
                            (pinf16, pinf16, big16) * _UNROLL)

        def merge(a, bb):
            v1 = jnp.minimum(a[0], bb[0])
            j1 = jnp.minimum(jnp.where(a[0] == v1, a[2], big16),
                             jnp.where(bb[0] == v1, bb[2], big16))
            ea = (a[0] == v1) & (a[2] == j1)
            eb = (bb[0] == v1) & (bb[2] == j1)
            v2 = jnp.minimum(jnp.where(ea, a[1], a[0]),
                             jnp.where(eb, bb[1], bb[0]))
            return v1, v2, j1

        m0 = merge(acc[0:3], acc[3:6])
        m1 = merge(acc[6:9], acc[9:12])
        v1l, v2l, j1l = merge(m0, m1)

        gv1 = jnp.min(v1l)
        gbj = jnp.min(jnp.where(v1l == gv1, j1l, big16))
        gv2 = jnp.min(jnp.where(j1l == gbj, v2l, v1l))
        incr = (gv2 - gv1) + eps

        gbj_v = jnp.broadcast_to(gbj, (16,))
        rid_v = jnp.broadcast_to(rid, (16,))
        dval = plsc.load_gather(buf, [gbj_v])
        plsc.store_scatter(pend_v, [rid_v], dval, mask=lane0)
        cur = plsc.load_gather(maxi_v, [gbj_v])
        incr_v = jnp.broadcast_to(incr, (16,))
        upd = (incr_v > cur) & lane0
        plsc.store_scatter(maxi_v, [gbj_v], incr_v, mask=upd)
        plsc.store_scatter(win_v, [gbj_v], rid_v, mask=upd)

    def _iter(t, carry):
        def ph_a1(g, carry1):
            for u in range(_UNROLL):
                ds = pl.ds(g * (16 * _UNROLL) + u * 16, 16)
                maxi_v[ds] = ninf16
                win_v[ds] = neg1
            return carry1

        lax.fori_loop(0, _CHUNKS // _UNROLL, ph_a1, 0)

        def ph_a2(g, cnt):
            for u in range(_UNROLL):
                base = hbase + g * (16 * _UNROLL) + u * 16
                ds = pl.ds(base, 16)
                mask = asg_v[ds] < 0
                pos = plsc.cumsum(mask.astype(jnp.int32))
                plsc.store_scatter(idl_v, [cnt + pos - 1], lanes + base,
                                   mask=mask)
                cnt = cnt + pos[15]
            return cnt

        nb = lax.fori_loop(0, _HCHUNKS // _UNROLL, ph_a2, jnp.int32(0))

        for u in range(_NBUF):
            @pl.when(u < nb)
            def _(u=u):
                _start(jnp.int32(u), bufs[u], sems[u])

        def ph_b(k, carry2):
            for u in range(_NBUF):
                q = k * _NBUF + u

                @pl.when(q < nb)
                def _(q=q, u=u):
                    _wait(bufs[u], sems[u])
                    _process(bufs[u], _get_rid(q))

                    @pl.when(q + _NBUF < nb)
                    def _(q=q, u=u):
                        _start(q + _NBUF, bufs[u], sems[u])

            return carry2

        lax.fori_loop(0, (nb + _NBUF - 1) // _NBUF, ph_b, 0)

        slot = (sl * 3 + h) * _N
        pltpu.sync_copy(maxi_v, shf.at[pl.ds(slot, _N)])
        pltpu.sync_copy(win_v, shi.at[pl.ds(slot, _N)])
        plsc.subcore_barrier()

        @pl.when(h == 0)
        def _master():
            odd = (sl * 3 + 1) * _N
            pltpu.sync_copy(shf.at[pl.ds(odd, _N)], mx1_v)
            pltpu.sync_copy(shi.at[pl.ds(odd, _N)], wn1_v)

            def ph_c(g, carry3):
                for u in range(_UNROLL):
                    base = g * (16 * _UNROLL) + u * 16
                    ds = pl.ds(base, 16)
                    mi0 = maxi_v[ds]
                    mi1 = mx1_v[ds]
                    take = mi1 > mi0
                    mi = jnp.where(take, mi1, mi0)
                    wn = jnp.where(take, wn1_v[ds], win_v[ds])
                    hb = mi > _NEG_INF
                    pr = price_v[ds]
                    price_v[ds] = jnp.where(hb, pr + mi, pr)
                    prev = inv_v[ds]
                    mask_e = hb & (prev >= 0)
                    plsc.store_scatter(asg_v, [jnp.where(mask_e, prev, 0)],
                                       neg1, mask=mask_e)
                    plsc.store_scatter(asg_v, [jnp.where(hb, wn, 0)],
                                       lanes + base, mask=hb)
                    inv_v[ds] = jnp.where(hb, wn, prev)
                return carry3

            lax.fori_loop(0, _CHUNKS // _UNROLL, ph_c, 0)
            can = (sl * 3 + 2) * _N
            pltpu.sync_copy(price_v, shf.at[pl.ds(can, _N)])
            pltpu.sync_copy(asg_v, shi.at[pl.ds(can, _N)])

        plsc.subcore_barrier()

        @pl.when(h == 1)
        def _slave():
            can = (sl * 3 + 2) * _N
            pltpu.sync_copy(shf.at[pl.ds(can, _N)], price_v)
            pltpu.sync_copy(shi.at[pl.ds(can, _N)], asg_v)

        return carry

    lax.fori_loop(0, niter, _iter, 0)

    def _fin(g, carry):
        for u in range(_UNROLL):
            ds = pl.ds(hbase + g * (16 * _UNROLL) + u * 16, 16)
            pend_v[ds] = jnp.where(asg_v[ds] >= 0, pend_v[ds], 0.0)
        return carry

    lax.fori_loop(0, _HCHUNKS // _UNROLL, _fin, 0)
    pltpu.sync_copy(pend_v.at[pl.ds(hbase, _HN)],
                    dist_hbm.at[pl.ds(rowbase + hbase, _HN)])

    @pl.when(h == 0)
    def _out_asg():
        pltpu.sync_copy(asg_v, asg_hbm.at[pl.ds(rowbase, _N)])


def kernel(input1, input2, eps, iters):
    B, n, _ = input1.shape
    m = input2.shape[1]
    s1 = jnp.sum(input1 * input1, axis=-1)[:, :, None]
    s2 = jnp.sum(input2 * input2, axis=-1)[:, None, :]
    cross = jnp.einsum('bnd,bmd->bnm', input1, input2)
    d2 = jnp.maximum(s1 + s2 - 2.0 * cross, 0.0)

    eps_arr = jnp.full((16,), eps, jnp.float32)
    it_arr = jnp.full((16,), iters, jnp.int32)
    mesh = plsc.VectorSubcoreMesh(core_axis_name="c", subcore_axis_name="s",
                                  num_cores=2, num_subcores=16)
    run = pl.kernel(
        _sc_body,
        out_type=[
            jax.ShapeDtypeStruct((B * n,), jnp.float32),
            jax.ShapeDtypeStruct((B * n,), jnp.int32),
        ],
        mesh=mesh,
        compiler_params=pltpu.CompilerParams(needs_layout_passes=False),
        scratch_types=[
            pltpu.VMEM((n,), jnp.float32),
            pltpu.VMEM((n,), jnp.int32),
            pltpu.VMEM((n,), jnp.int32),
            pltpu.VMEM((n,), jnp.float32),
            pltpu.VMEM((n,), jnp.int32),
            pltpu.VMEM((n,), jnp.float32),
            pltpu.VMEM((n,), jnp.int32),
            pltpu.VMEM((n,), jnp.float32),
            pltpu.VMEM((n,), jnp.int32),
            pltpu.VMEM((m,), jnp.float32),
            pltpu.VMEM((m,), jnp.float32),
            pltpu.VMEM((m,), jnp.float32),
            pltpu.VMEM((m,), jnp.float32),
            pltpu.VMEM((16,), jnp.float32),
            pltpu.VMEM((16,), jnp.int32),
            pltpu.VMEM_SHARED((8 * 3 * n,), jnp.float32),
            pltpu.VMEM_SHARED((8 * 3 * n,), jnp.int32),
            pltpu.SemaphoreType.DMA,
            pltpu.SemaphoreType.DMA,
            pltpu.SemaphoreType.DMA,
            pltpu.SemaphoreType.DMA,
        ],
    )
    dist, asg = run(d2.reshape(B * n, m), eps_arr, it_arr)
    return dist.reshape(B, n), asg.reshape(B, n)


# speedup vs baseline: 4.2784x; 1.1277x over previous
import jax
import jax.numpy as jnp
from jax import lax
from jax.experimental import pallas as pl
from jax.experimental.pallas import tpu as pltpu
from jax.experimental.pallas import tpu_sc as plsc

_NEG_INF = float("-inf")
_POS_INF = float("inf")
_N = 1024
_HN = _N // 2
_CHUNKS = _N // 16
_HCHUNKS = _HN // 16
_UNROLL = 4
_NBUF = 6


def _sc_body(d2_hbm, eps_hbm, it_hbm, dist_hbm, asg_hbm,
             price_v, asg_v, inv_v, maxi_v, win_v, pend_v, idl_v,
             mx1_v, wn1_v,
             row_0, row_1, row_2, row_3, row_4, row_5, scal_f, scal_i,
             shf, shi,
             sem_0, sem_1, sem_2, sem_3, sem_4, sem_5):
    c = lax.axis_index("c")
    s = lax.axis_index("s")
    sl = s // 2
    h = s % 2
    b = c * 8 + sl
    bufs = (row_0, row_1, row_2, row_3, row_4, row_5)
    sems = (sem_0, sem_1, sem_2, sem_3, sem_4, sem_5)

    pltpu.sync_copy(eps_hbm, scal_f)
    pltpu.sync_copy(it_hbm, scal_i)
    eps = scal_f[...][0]
    niter = scal_i[...][0]

    lanes = lax.iota(jnp.int32, 16)
    zf16 = jnp.zeros((16,), jnp.float32)
    neg1 = jnp.full((16,), -1, jnp.int32)
    ninf16 = jnp.full((16,), _NEG_INF, jnp.float32)
    pinf16 = jnp.full((16,), _POS_INF, jnp.float32)
    big16 = jnp.full((16,), 1 << 30, jnp.int32)
    lane0 = lanes == 0
    rowbase = b * _N
    hbase = h * _HN

    def _init(g, carry):
        for u in range(_UNROLL):
            ds = pl.ds(g * (16 * _UNROLL) + u * 16, 16)
            price_v[ds] = zf16
            asg_v[ds] = neg1
            inv_v[ds] = neg1
            pend_v[ds] = zf16
        return carry

    lax.fori_loop(0, _CHUNKS // _UNROLL, _init, 0)

    def _get_rid(q):
        qv = jnp.broadcast_to(q, (16,))
        return plsc.load_gather(idl_v, [qv])[0]

    def _start(q, buf, sem):
        src = d2_hbm.at[rowbase + _get_rid(q)]
        pltpu.make_async_copy(src, buf, sem).start()

    def _wait(buf, sem):
        pltpu.make_async_copy(d2_hbm.at[0], buf, sem).wait()

    def _process(buf, rid):
        def scan_c(g, car):
            out = []
            base = g * (16 * _UNROLL)
            for u in range(_UNROLL):
                v1l, v2l, j1l = car[3 * u:3 * u + 3]
                ds = pl.ds(base + u * 16, 16)
                v = buf[ds] + price_v[ds]
                cols = lanes + (base + u * 16)
                lt1 = v < v1l
                v2l = jnp.minimum(v2l, jnp.maximum(v1l, v))
                v1l = jnp.minimum(v1l, v)
                j1l = jnp.where(lt1, cols, j1l)
                out += [v1l, v2l, j1l]
            return tuple(out)

        acc = lax.fori_loop(0, _CHUNKS // _UNROLL, scan_c,
                            (pinf16, pinf16, big16) * _UNROLL)

        def merge(a, bb):
            v1 = jnp.minimum(a[0], bb[0])
            j1 = jnp.minimum(jnp.where(a[0] == v1, a[2], big16),
                             jnp.where(bb[0] == v1, bb[2], big16))
            ea = (a[0] == v1) & (a[2] == j1)
            eb = (bb[0] == v1) & (bb[2] == j1)
            v2 = jnp.minimum(jnp.where(ea, a[1], a[0]),
                             jnp.where(eb, bb[1], bb[0]))
            return v1, v2, j1

        m0 = merge(acc[0:3], acc[3:6])
        m1 = merge(acc[6:9], acc[9:12])
        v1l, v2l, j1l = merge(m0, m1)

        gv1 = jnp.min(v1l)
        gbj = jnp.min(jnp.where(v1l == gv1, j1l, big16))
        gv2 = jnp.min(jnp.where(j1l == gbj, v2l, v1l))
        incr = (gv2 - gv1) + eps

        gbj_v = jnp.broadcast_to(gbj, (16,))
        rid_v = jnp.broadcast_to(rid, (16,))
        dval = plsc.load_gather(buf, [gbj_v])
        plsc.store_scatter(pend_v, [rid_v], dval, mask=lane0)
        cur = plsc.load_gather(maxi_v, [gbj_v])
        incr_v = jnp.broadcast_to(incr, (16,))
        upd = (incr_v > cur) & lane0
        plsc.store_scatter(maxi_v, [gbj_v], incr_v, mask=upd)
        plsc.store_scatter(win_v, [gbj_v], rid_v, mask=upd)

    def _iter(t, carry):
        def ph_a1(g, carry1):
            for u in range(_UNROLL):
                ds = pl.ds(g * (16 * _UNROLL) + u * 16, 16)
                maxi_v[ds] = ninf16
                win_v[ds] = neg1
            return carry1

        lax.fori_loop(0, _CHUNKS // _UNROLL, ph_a1, 0)

        def ph_a2(g, cnt):
            for u in range(_UNROLL):
                base = hbase + g * (16 * _UNROLL) + u * 16
                ds = pl.ds(base, 16)
                mask = asg_v[ds] < 0
                pos = plsc.cumsum(mask.astype(jnp.int32))
                plsc.store_scatter(idl_v, [cnt + pos - 1], lanes + base,
                                   mask=mask)
                cnt = cnt + pos[15]
            return cnt

        nb = lax.fori_loop(0, _HCHUNKS // _UNROLL, ph_a2, jnp.int32(0))

        for u in range(_NBUF):
            @pl.when(u < nb)
            def _(u=u):
                _start(jnp.int32(u), bufs[u], sems[u])

        def ph_b(k, carry2):
            for u in range(_NBUF):
                q = k * _NBUF + u

                @pl.when(q < nb)
                def _(q=q, u=u):
                    _wait(bufs[u], sems[u])
                    _process(bufs[u], _get_rid(q))

                    @pl.when(q + _NBUF < nb)
                    def _(q=q, u=u):
                        _start(q + _NBUF, bufs[u], sems[u])

            return carry2

        lax.fori_loop(0, (nb + _NBUF - 1) // _NBUF, ph_b, 0)

        slot = (sl * 3 + h) * _N
        pltpu.sync_copy(maxi_v, shf.at[pl.ds(slot, _N)])
        pltpu.sync_copy(win_v, shi.at[pl.ds(slot, _N)])
        plsc.subcore_barrier()

        @pl.when(h == 0)
        def _master():
            odd = (sl * 3 + 1) * _N
            pltpu.sync_copy(shf.at[pl.ds(odd, _N)], mx1_v)
            pltpu.sync_copy(shi.at[pl.ds(odd, _N)], wn1_v)

            def ph_c(g, carry3):
                for u in range(_UNROLL):
                    base = g * (16 * _UNROLL) + u * 16
                    ds = pl.ds(base, 16)
                    mi0 = maxi_v[ds]
                    mi1 = mx1_v[ds]
                    take = mi1 > mi0
                    mi = jnp.where(take, mi1, mi0)
                    wn = jnp.where(take, wn1_v[ds], win_v[ds])
                    hb = mi > _NEG_INF
                    pr = price_v[ds]
                    price_v[ds] = jnp.where(hb, pr + mi, pr)
                    prev = inv_v[ds]
                    mask_e = hb & (prev >= 0)
                    plsc.store_scatter(asg_v, [jnp.where(mask_e, prev, 0)],
                                       neg1, mask=mask_e)
                    plsc.store_scatter(asg_v, [jnp.where(hb, wn, 0)],
                                       lanes + base, mask=hb)
                    inv_v[ds] = jnp.where(hb, wn, prev)
                return carry3

            lax.fori_loop(0, _CHUNKS // _UNROLL, ph_c, 0)
            can = (sl * 3 + 2) * _N
            pltpu.sync_copy(price_v, shf.at[pl.ds(can, _N)])
            pltpu.sync_copy(asg_v, shi.at[pl.ds(can, _N)])

        plsc.subcore_barrier()

        @pl.when(h == 1)
        def _slave():
            can = (sl * 3 + 2) * _N
            pltpu.sync_copy(shf.at[pl.ds(can, _N)], price_v)
            pltpu.sync_copy(shi.at[pl.ds(can, _N)], asg_v)

        return carry

    lax.fori_loop(0, niter, _iter, 0)

    def _fin(g, carry):
        for u in range(_UNROLL):
            ds = pl.ds(hbase + g * (16 * _UNROLL) + u * 16, 16)
            pend_v[ds] = jnp.where(asg_v[ds] >= 0, pend_v[ds], 0.0)
        return carry

    lax.fori_loop(0, _HCHUNKS // _UNROLL, _fin, 0)
    pltpu.sync_copy(pend_v.at[pl.ds(hbase, _HN)],
                    dist_hbm.at[pl.ds(rowbase + hbase, _HN)])

    @pl.when(h == 0)
    def _out_asg():
        pltpu.sync_copy(asg_v, asg_hbm.at[pl.ds(rowbase, _N)])


def kernel(input1, input2, eps, iters):
    B, n, _ = input1.shape
    m = input2.shape[1]
    s1 = jnp.sum(input1 * input1, axis=-1)[:, :, None]
    s2 = jnp.sum(input2 * input2, axis=-1)[:, None, :]
    cross = jnp.einsum('bnd,bmd->bnm', input1, input2)
    d2 = jnp.maximum(s1 + s2 - 2.0 * cross, 0.0)

    eps_arr = jnp.full((16,), eps, jnp.float32)
    it_arr = jnp.full((16,), iters, jnp.int32)
    mesh = plsc.VectorSubcoreMesh(core_axis_name="c", subcore_axis_name="s",
                                  num_cores=2, num_subcores=16)
    run = pl.kernel(
        _sc_body,
        out_type=[
            jax.ShapeDtypeStruct((B * n,), jnp.float32),
            jax.ShapeDtypeStruct((B * n,), jnp.int32),
        ],
        mesh=mesh,
        compiler_params=pltpu.CompilerParams(needs_layout_passes=False),
        scratch_types=[
            pltpu.VMEM((n,), jnp.float32),
            pltpu.VMEM((n,), jnp.int32),
            pltpu.VMEM((n,), jnp.int32),
            pltpu.VMEM((n,), jnp.float32),
            pltpu.VMEM((n,), jnp.int32),
            pltpu.VMEM((n,), jnp.float32),
            pltpu.VMEM((n,), jnp.int32),
            pltpu.VMEM((n,), jnp.float32),
            pltpu.VMEM((n,), jnp.int32),
            pltpu.VMEM((m,), jnp.float32),
            pltpu.VMEM((m,), jnp.float32),
            pltpu.VMEM((m,), jnp.float32),
            pltpu.VMEM((m,), jnp.float32),
            pltpu.VMEM((m,), jnp.float32),
            pltpu.VMEM((m,), jnp.float32),
            pltpu.VMEM((16,), jnp.float32),
            pltpu.VMEM((16,), jnp.int32),
            pltpu.VMEM_SHARED((8 * 3 * n,), jnp.float32),
            pltpu.VMEM_SHARED((8 * 3 * n,), jnp.int32),
            pltpu.SemaphoreType.DMA,
            pltpu.SemaphoreType.DMA,
            pltpu.SemaphoreType.DMA,
            pltpu.SemaphoreType.DMA,
            pltpu.SemaphoreType.DMA,
            pltpu.SemaphoreType.DMA,
        ],
    )
    dist, asg = run(d2.reshape(B * n, m), eps_arr, it_arr)
    return dist.reshape(B, n), asg.reshape(B, n)
